# baseline (speedup 1.0000x reference)
import functools

import jax
import jax.numpy as jnp
from jax import lax
from jax.experimental import pallas as pl
from jax.experimental.pallas import tpu as pltpu
from jax.experimental.pallas import tpu_sc as plsc

_NUM_PATCHES = 1024
_DIM = 768
_NC = 2
_NS = 16
_NW = _NC * _NS
_ROWS = _NUM_PATCHES // _NW
_B1 = 40


def _make_sc_bcast(batch):
    mesh = plsc.VectorSubcoreMesh(core_axis_name="c", subcore_axis_name="s")
    spmem_bats = (batch - _B1) // _NC

    @functools.partial(
        pl.kernel,
        mesh=mesh,
        out_type=jax.ShapeDtypeStruct((batch, _NUM_PATCHES, _DIM), jnp.float32),
        scratch_types=[
            pltpu.VMEM((_ROWS, _DIM), jnp.float32),
            pltpu.VMEM_SHARED((_NUM_PATCHES, _DIM), jnp.float32),
            pltpu.SemaphoreType.DMA,
            pltpu.SemaphoreType.DMA,
        ],
    )
    def sc_bcast(table_hbm, out_hbm, chunk_v, tab_sp, sem, sp_sem):
        cid = lax.axis_index("c")
        sid = lax.axis_index("s")
        wid = sid * _NC + cid
        base = wid * _ROWS
        pltpu.sync_copy(table_hbm.at[pl.ds(base, _ROWS)], chunk_v)

        @pl.when(sid == 0)
        def _spmem_fire():
            pltpu.sync_copy(table_hbm, tab_sp)
            b0 = _B1 + cid * spmem_bats
            for j in range(spmem_bats):
                pltpu.make_async_copy(tab_sp, out_hbm.at[b0 + j], sp_sem).start()

        def fire(b, carry):
            pltpu.make_async_copy(
                chunk_v, out_hbm.at[b, pl.ds(base, _ROWS)], sem
            ).start()
            return carry

        lax.fori_loop(0, _B1, fire, 0)

        def drain(b, carry):
            pltpu.make_async_copy(
                chunk_v, out_hbm.at[0, pl.ds(base, _ROWS)], sem
            ).wait()
            return carry

        lax.fori_loop(0, _B1, drain, 0)

        @pl.when(sid == 0)
        def _spmem_drain():
            for _ in range(spmem_bats):
                pltpu.make_async_copy(tab_sp, out_hbm.at[0], sp_sem).wait()

    return sc_bcast


def kernel(x, table):
    return _make_sc_bcast(x.shape[0])(table)
